# baseline (speedup 1.0000x reference)
import functools

import jax
import jax.numpy as jnp
from jax import lax
from jax.experimental import pallas as pl
from jax.experimental.pallas import tpu as pltpu
from jax.experimental.pallas import tpu_sc as plsc

NUM_BUCKETS = 32
N_HEADS = 16
SEQ = 2048
STRIP = 2 * SEQ

NUM_CORES = 2
NUM_SUBCORES = 16
LANES = 16

QT_PER_HEAD = SEQ // 8
QT_PER_WORKER = QT_PER_HEAD // NUM_CORES

J_LO = 1792 // LANES
J_HI = 2432 // LANES


def _make_fill():
    mesh = plsc.VectorSubcoreMesh(core_axis_name="c", subcore_axis_name="s")

    @functools.partial(
        pl.kernel,
        mesh=mesh,
        out_type=jax.ShapeDtypeStruct((N_HEADS, QT_PER_HEAD, 16, 8, 128), jnp.float32),
        scratch_types=[
            pltpu.VMEM((NUM_BUCKETS, N_HEADS), jnp.float32),
            pltpu.VMEM((8 * STRIP,), jnp.float32),
            pltpu.VMEM((4, 2, 8, 128), jnp.float32),
            pltpu.VMEM((14, 8, 128), jnp.float32),
            pltpu.VMEM((14, 8, 128), jnp.float32),
            pltpu.VMEM((16, 8, 128), jnp.float32),
            pltpu.SemaphoreType.DMA,
        ],
        compiler_params=pltpu.CompilerParams(needs_layout_passes=False),
    )
    def fill(table_hbm, out_hbm, table_v, strips_v, tr_v, c15_v, c31_v, drain_v, sem):
        head = lax.axis_index("s")
        half = lax.axis_index("c")

        pltpu.sync_copy(table_hbm, table_v)

        head_idx = jnp.full((LANES,), head, dtype=jnp.int32)
        lane = lax.broadcasted_iota(jnp.int32, (LANES,), 0)

        for r in range(8):

            def strip_body(j, carry, r=r):
                doff = j * LANES + lane + r
                d = doff - (SEQ - 1)
                n = jnp.minimum(jnp.abs(d), 15)
                b = n + 16 * (d > 0).astype(jnp.int32)
                vals = plsc.load_gather(table_v, [b, head_idx])
                strips_v[pl.ds(r * STRIP + j * LANES, LANES)] = vals
                return carry

            lax.fori_loop(J_LO, J_HI, strip_body, 0)

        v15 = plsc.load_gather(table_v, [jnp.full((LANES,), 15, jnp.int32), head_idx])
        v31 = plsc.load_gather(table_v, [jnp.full((LANES,), 31, jnp.int32), head_idx])

        def const_body(ci, carry):
            for s in range(8):
                for t in range(8):
                    c15_v[ci, s, pl.ds(16 * t, LANES)] = v15
                    c31_v[ci, s, pl.ds(16 * t, LANES)] = v31
            return carry

        lax.fori_loop(0, 14, const_body, 0)

        qt0 = half * QT_PER_WORKER

        def drain():
            pltpu.make_async_copy(out_hbm.at[0, 0], drain_v, sem).wait()

        def emit(buf, qt):
            i0 = (SEQ - 8) - 8 * qt
            ca = jnp.minimum(jnp.maximum((8 * qt - 14) // 128, 0), 14)

            cbase = i0 + 128 * ca
            for ci in range(2):
                for s in range(8):
                    r = 7 - s
                    for t in range(8):
                        vals = strips_v[
                            pl.ds(cbase + (r * STRIP + 128 * ci + 16 * t), LANES)
                        ]
                        tr_v[buf, ci, s, pl.ds(16 * t, LANES)] = vals

            for K in range(15):

                @pl.when(ca == K)
                def _(K=K):
                    if K > 0:
                        pltpu.async_copy(
                            c15_v.at[pl.ds(0, K)],
                            out_hbm.at[head, qt, pl.ds(0, K)],
                            sem,
                        )
                    pltpu.async_copy(
                        tr_v.at[buf], out_hbm.at[head, qt, pl.ds(K, 2)], sem
                    )
                    if K < 14:
                        pltpu.async_copy(
                            c31_v.at[pl.ds(0, 14 - K)],
                            out_hbm.at[head, qt, pl.ds(K + 2, 14 - K)],
                            sem,
                        )

        def quad_body(g, carry):
            for b in range(4):

                @pl.when(g > 0)
                def _():
                    drain()

                emit(b, qt0 + 4 * g + b)
            return carry

        lax.fori_loop(0, QT_PER_WORKER // 4, quad_body, 0)
        for _ in range(4):
            drain()

    return fill


_fill = _make_fill()


def kernel(relative_attention_bias, qlen, klen):
    del qlen, klen
    t5 = _fill(relative_attention_bias)
    return jnp.transpose(t5, (0, 1, 3, 2, 4)).reshape(1, N_HEADS, SEQ, SEQ)


# speedup vs baseline: 1.0130x; 1.0130x over previous
import functools

import jax
import jax.numpy as jnp
from jax import lax
from jax.experimental import pallas as pl
from jax.experimental.pallas import tpu as pltpu
from jax.experimental.pallas import tpu_sc as plsc

NUM_BUCKETS = 32
N_HEADS = 16
SEQ = 2048
STRIP = 2 * SEQ

NUM_CORES = 2
NUM_SUBCORES = 16
LANES = 16

QT_PER_HEAD = SEQ // 8
QT_PER_WORKER = QT_PER_HEAD // NUM_CORES

J_LO = 1792 // LANES
J_HI = 2432 // LANES


def _make_fill():
    mesh = plsc.VectorSubcoreMesh(core_axis_name="c", subcore_axis_name="s")

    @functools.partial(
        pl.kernel,
        mesh=mesh,
        out_type=jax.ShapeDtypeStruct((N_HEADS, QT_PER_HEAD, 16, 8, 128), jnp.float32),
        scratch_types=[
            pltpu.VMEM((NUM_BUCKETS, N_HEADS), jnp.float32),
            pltpu.VMEM((8 * STRIP,), jnp.float32),
            pltpu.VMEM((2, 2, 8, 128), jnp.float32),
            pltpu.VMEM((14, 8, 128), jnp.float32),
            pltpu.VMEM((14, 8, 128), jnp.float32),
            pltpu.VMEM((16, 8, 128), jnp.float32),
            pltpu.SemaphoreType.DMA,
        ],
        compiler_params=pltpu.CompilerParams(needs_layout_passes=False),
    )
    def fill(table_hbm, out_hbm, table_v, strips_v, tr_v, c15_v, c31_v, drain_v, sem):
        head = lax.axis_index("s")
        half = lax.axis_index("c")

        pltpu.sync_copy(table_hbm, table_v)

        head_idx = jnp.full((LANES,), head, dtype=jnp.int32)
        lane = lax.broadcasted_iota(jnp.int32, (LANES,), 0)

        for r in range(8):

            def strip_body(j, carry, r=r):
                doff = j * LANES + lane + r
                d = doff - (SEQ - 1)
                n = jnp.minimum(jnp.abs(d), 15)
                b = n + 16 * (d > 0).astype(jnp.int32)
                vals = plsc.load_gather(table_v, [b, head_idx])
                strips_v[pl.ds(r * STRIP + j * LANES, LANES)] = vals
                return carry

            lax.fori_loop(J_LO, J_HI, strip_body, 0)

        v15 = plsc.load_gather(table_v, [jnp.full((LANES,), 15, jnp.int32), head_idx])
        v31 = plsc.load_gather(table_v, [jnp.full((LANES,), 31, jnp.int32), head_idx])

        def const_body(ci, carry):
            for s in range(8):
                for t in range(8):
                    c15_v[ci, s, pl.ds(16 * t, LANES)] = v15
                    c31_v[ci, s, pl.ds(16 * t, LANES)] = v31
            return carry

        lax.fori_loop(0, 14, const_body, 0)

        qt0 = half * QT_PER_WORKER

        def drain():
            pltpu.make_async_copy(out_hbm.at[0, 0], drain_v, sem).wait()

        def emit(buf, qt):
            i0 = (SEQ - 8) - 8 * qt
            ca = jnp.minimum(jnp.maximum((8 * qt - 14) // 128, 0), 14)

            cbase = i0 + 128 * ca
            for ci in range(2):
                for s in range(8):
                    r = 7 - s
                    for t in range(8):
                        vals = strips_v[
                            pl.ds(cbase + (r * STRIP + 128 * ci + 16 * t), LANES)
                        ]
                        tr_v[buf, ci, s, pl.ds(16 * t, LANES)] = vals

            for K in range(15):

                @pl.when(ca == K)
                def _(K=K):
                    if K > 0:
                        pltpu.async_copy(
                            c15_v.at[pl.ds(0, K)],
                            out_hbm.at[head, qt, pl.ds(0, K)],
                            sem,
                        )
                    pltpu.async_copy(
                        tr_v.at[buf], out_hbm.at[head, qt, pl.ds(K, 2)], sem
                    )
                    if K < 14:
                        pltpu.async_copy(
                            c31_v.at[pl.ds(0, 14 - K)],
                            out_hbm.at[head, qt, pl.ds(K + 2, 14 - K)],
                            sem,
                        )

        def pair_body(g, carry):
            for b in range(2):

                @pl.when(g > 0)
                def _():
                    drain()

                emit(b, qt0 + 2 * g + b)
            return carry

        lax.fori_loop(0, QT_PER_WORKER // 2, pair_body, 0)
        drain()
        drain()

    return fill


_fill = _make_fill()


def kernel(relative_attention_bias, qlen, klen):
    del qlen, klen
    t5 = _fill(relative_attention_bias)
    return jnp.transpose(t5, (0, 1, 3, 2, 4)).reshape(1, N_HEADS, SEQ, SEQ)
